# speedup vs baseline: 3.4219x; 3.4219x over previous
import functools

import jax
import jax.numpy as jnp
from jax import lax
from jax.experimental import pallas as pl
from jax.experimental.pallas import tpu as pltpu
from jax.experimental.pallas import tpu_sc as plsc

_NC = 2
_NS = 16
_NW = _NC * _NS
_L = 16

_R = 128
_DEPTH = 8


def _build_scatter(rows_per_w: int, in_features: int):
  mesh = plsc.VectorSubcoreMesh(core_axis_name="c", subcore_axis_name="s")

  @functools.partial(
      pl.kernel,
      mesh=mesh,
      out_type=[],
      scratch_types=[
          pltpu.VMEM((rows_per_w, _R), jnp.int32),
          pltpu.VMEM((rows_per_w, _R), jnp.int32),
          pltpu.VMEM((rows_per_w, _R), jnp.float32),
          pltpu.SemaphoreType.DMA,
      ],
  )
  def scatter_kernel(rows_hbm, cols_hbm, vals_hbm, w_hbm, rbuf, ibuf, vbuf,
                     sem):
    wid = lax.axis_index("s") * _NC + lax.axis_index("c")
    pltpu.sync_copy(rows_hbm.at[wid], rbuf)
    pltpu.sync_copy(cols_hbm.at[wid], ibuf)
    pltpu.sync_copy(vals_hbm.at[wid], vbuf)

    depth = min(_DEPTH, rows_per_w)

    @pl.loop(0, rows_per_w)
    def _row(j):
      for k in range(_R // _L):
        sl = pl.ds(k * _L, _L)
        ibuf[j, sl] = rbuf[j, sl] * in_features + ibuf[j, sl]
      pltpu.async_copy(vbuf.at[j], w_hbm.at[ibuf.at[j]], sem)

      @pl.when(j >= depth)
      def _():
        pltpu.make_async_copy(
            vbuf.at[j - depth], w_hbm.at[ibuf.at[j - depth]], sem).wait()

    @pl.loop(rows_per_w - depth, rows_per_w)
    def _tail(j):
      pltpu.make_async_copy(vbuf.at[j], w_hbm.at[ibuf.at[j]], sem).wait()

  return scatter_kernel


def _mm_body(x_ref, w_ref, b_ref, o_ref):
  acc = lax.dot_general(x_ref[...], w_ref[...], (((1,), (1,)), ((), ())))
  o_ref[...] = acc + b_ref[...]


def _matmul(x, w2d, bias2d, bm: int, bn: int):
  batch, in_features = x.shape
  out_features = bias2d.shape[1]
  grid = (batch // bm, out_features // bn)
  return pl.pallas_call(
      _mm_body,
      grid=grid,
      in_specs=[
          pl.BlockSpec((bm, in_features), lambda i, j: (i, 0)),
          pl.BlockSpec((bn, in_features), lambda i, j: (j, 0)),
          pl.BlockSpec((1, bn), lambda i, j: (0, j)),
      ],
      out_specs=pl.BlockSpec((bm, bn), lambda i, j: (i, j)),
      out_shape=jax.ShapeDtypeStruct((batch, out_features), jnp.float32),
  )(x, w2d, bias2d)


def kernel(inputs, indices, active_fwd_weights, bias):
  batch, in_features = inputs.shape
  out_features = bias.shape[0]
  nnz = indices.shape[1]

  rows_per_w = -(-nnz // (_NW * _R))
  padded = _NW * rows_per_w * _R
  pad = padded - nnz

  rows = jnp.concatenate(
      [indices[0], jnp.full((pad,), out_features, jnp.int32)])
  cols = jnp.concatenate([indices[1], jnp.zeros((pad,), jnp.int32)])
  vals = jnp.concatenate(
      [active_fwd_weights, jnp.zeros((pad,), jnp.float32)])
  rows3 = rows.reshape(_NW, rows_per_w, _R)
  cols3 = cols.reshape(_NW, rows_per_w, _R)
  vals3 = vals.reshape(_NW, rows_per_w, _R)

  scatter = _build_scatter(rows_per_w, in_features)
  w_ref = jax.new_ref(
      jnp.zeros(((out_features + 1) * in_features,), jnp.float32))
  scatter(rows3, cols3, vals3, w_ref)
  w2d = w_ref[...].reshape(out_features + 1, in_features)

  return _matmul(inputs, w2d, bias.reshape(1, out_features), bm=512, bn=512)


# speedup vs baseline: 3.6147x; 1.0563x over previous
import functools

import jax
import jax.numpy as jnp
from jax import lax
from jax.experimental import pallas as pl
from jax.experimental.pallas import tpu as pltpu
from jax.experimental.pallas import tpu_sc as plsc

_NC = 2
_NS = 16
_L = 16

_R = 128
_SEC = 32
_ZCH = 4096


def _build_densify(rows_per_t: int, out_features: int, in_features: int):
  n_pass = 2
  win_rows = out_features // (_NC * n_pass)
  win = win_rows * in_features
  dump = win
  stripe = win // _NS
  n_sec = rows_per_t // _SEC

  mesh = plsc.VectorSubcoreMesh(core_axis_name="c", subcore_axis_name="s")

  @functools.partial(
      pl.kernel,
      mesh=mesh,
      out_type=jax.ShapeDtypeStruct((out_features * in_features,),
                                    jnp.float32),
      scratch_types=[
          pltpu.VMEM((2, _SEC, _R), jnp.int32),
          pltpu.VMEM((2, _SEC, _R), jnp.int32),
          pltpu.VMEM((2, _SEC, _R), jnp.float32),
          pltpu.VMEM((2, _SEC, _R), jnp.int32),
          pltpu.VMEM((_ZCH,), jnp.float32),
          pltpu.VMEM_SHARED((win + 8,), jnp.float32),
          pltpu.SemaphoreType.DMA,
          pltpu.SemaphoreType.DMA,
          pltpu.SemaphoreType.DMA,
      ],
  )
  def densify(rows_hbm, cols_hbm, vals_hbm, w_hbm, rbuf, cbuf, vbuf, lbuf,
              zbuf, acc, sem, insem, zsem):
    c = lax.axis_index("c")
    s = lax.axis_index("s")

    def _fire_in(sec, st):
      pltpu.async_copy(rows_hbm.at[s, pl.ds(sec * _SEC, _SEC)],
                       rbuf.at[st], insem)
      pltpu.async_copy(cols_hbm.at[s, pl.ds(sec * _SEC, _SEC)],
                       cbuf.at[st], insem)
      pltpu.async_copy(vals_hbm.at[s, pl.ds(sec * _SEC, _SEC)],
                       vbuf.at[st], insem)

    def _wait_in(st):
      pltpu.make_async_copy(rows_hbm.at[s, pl.ds(0, _SEC)],
                            rbuf.at[st], insem).wait()
      pltpu.make_async_copy(cols_hbm.at[s, pl.ds(0, _SEC)],
                            cbuf.at[st], insem).wait()
      pltpu.make_async_copy(vals_hbm.at[s, pl.ds(0, _SEC)],
                            vbuf.at[st], insem).wait()

    def _drain_scatter(st):
      @pl.loop(0, _SEC)
      def _d(j):
        pltpu.make_async_copy(vbuf.at[st, j], acc.at[lbuf.at[st, j]],
                              sem).wait()

    @pl.loop(0, _ZCH // _L)
    def _z(i):
      zbuf[pl.ds(i * _L, _L)] = jnp.zeros((_L,), jnp.float32)

    n_z = stripe // _ZCH
    my0 = s * stripe

    def _fire_zero():
      @pl.loop(0, n_z)
      def _zf(i):
        pltpu.async_copy(zbuf, acc.at[pl.ds(my0 + i * _ZCH, _ZCH)], zsem)

    def _drain_zero():
      @pl.loop(0, n_z)
      def _zd(i):
        pltpu.make_async_copy(
            zbuf, acc.at[pl.ds(my0 + i * _ZCH, _ZCH)], zsem).wait()

    _fire_zero()
    _drain_zero()
    plsc.subcore_barrier()

    for p in range(n_pass):
      row_base = (c * n_pass + p) * win_rows

      _fire_in(0, 0)

      @pl.loop(0, n_sec)
      def _sec(i, row_base=row_base):
        cur = lax.rem(i, 2)
        nxt = 1 - cur

        @pl.when(i >= 1)
        def _():
          _drain_scatter(nxt)

        @pl.when(i + 1 < n_sec)
        def _():
          _fire_in(i + 1, nxt)

        _wait_in(cur)

        @pl.loop(0, _SEC)
        def _row(j, row_base=row_base):
          for k in range(_R // _L):
            sl = pl.ds(k * _L, _L)
            local = (rbuf[cur, j, sl] - row_base) * in_features \
                + cbuf[cur, j, sl]
            ok = local.astype(jnp.uint32) < jnp.uint32(win)
            lbuf[cur, j, sl] = jnp.where(ok, local, dump)
          pltpu.async_copy(vbuf.at[cur, j], acc.at[lbuf.at[cur, j]], sem,
                           add=True)

      _drain_scatter((n_sec - 1) % 2)
      plsc.subcore_barrier()

      pltpu.sync_copy(acc.at[pl.ds(my0, stripe)],
                      w_hbm.at[pl.ds(row_base * in_features + my0, stripe)])

      if p + 1 < n_pass:
        _fire_zero()
        _drain_zero()
        plsc.subcore_barrier()

  return densify


def _mm_body(x_ref, w_ref, b_ref, o_ref):
  acc = lax.dot_general(x_ref[...], w_ref[...], (((1,), (1,)), ((), ())))
  o_ref[...] = acc + b_ref[...]


def _matmul(x, w2d, bias2d, bm: int, bn: int):
  batch, in_features = x.shape
  out_features = bias2d.shape[1]
  grid = (batch // bm, out_features // bn)
  return pl.pallas_call(
      _mm_body,
      grid=grid,
      in_specs=[
          pl.BlockSpec((bm, in_features), lambda i, j: (i, 0)),
          pl.BlockSpec((bn, in_features), lambda i, j: (j, 0)),
          pl.BlockSpec((1, bn), lambda i, j: (0, j)),
      ],
      out_specs=pl.BlockSpec((bm, bn), lambda i, j: (i, j)),
      out_shape=jax.ShapeDtypeStruct((batch, out_features), jnp.float32),
  )(x, w2d, bias2d)


def kernel(inputs, indices, active_fwd_weights, bias):
  batch, in_features = inputs.shape
  out_features = bias.shape[0]
  nnz = indices.shape[1]

  rows_per_t = -(-nnz // (_NS * _R))
  rows_per_t = -(-rows_per_t // _SEC) * _SEC
  padded = _NS * rows_per_t * _R
  pad = padded - nnz

  rows = jnp.concatenate(
      [indices[0], jnp.full((pad,), out_features, jnp.int32)])
  cols = jnp.concatenate([indices[1], jnp.zeros((pad,), jnp.int32)])
  vals = jnp.concatenate(
      [active_fwd_weights, jnp.zeros((pad,), jnp.float32)])
  rows3 = rows.reshape(_NS, rows_per_t, _R)
  cols3 = cols.reshape(_NS, rows_per_t, _R)
  vals3 = vals.reshape(_NS, rows_per_t, _R)

  densify = _build_densify(rows_per_t, out_features, in_features)
  w_flat = densify(rows3, cols3, vals3)
  w2d = w_flat.reshape(out_features, in_features)

  return _matmul(inputs, w2d, bias.reshape(1, out_features), bm=512, bn=512)
